# Initial kernel scaffold; baseline (speedup 1.0000x reference)
import jax
import jax.numpy as jnp
from jax.experimental import pallas as pl


def kernel(customer_emb, product_emb, edge_index, W1, b1, W2, b2, W3, b3):
    raise NotImplementedError("write your pallas kernel here")


# speedup vs baseline: 3.7215x; 3.7215x over previous
import functools

import jax
import jax.numpy as jnp
from jax import lax
from jax.experimental import pallas as pl
from jax.experimental.pallas import tpu as pltpu
from jax.experimental.pallas import tpu_sc as plsc

N_NODES = 10000
N_EDGES = 320000
EMB = 160
H1 = 32
H2 = 16

_INFO = plsc.get_sparse_core_info()
_NC, _NS, _NL = _INFO.num_cores, _INFO.num_subcores, _INFO.num_lanes
_NW = _NC * _NS
_CHUNK = 128
_NCHUNK = N_EDGES // _CHUNK
_ITERS = -(-_NCHUNK // _NW)


def _precompute_body(cust_ref, prod_ref, w1_ref, b1_ref, a_ref, b_ref):
    w_top = w1_ref[0:EMB, :]
    w_bot = w1_ref[EMB : 2 * EMB, :]
    a_ref[...] = (
        jnp.dot(cust_ref[...], w_top, preferred_element_type=jnp.float32)
        + b1_ref[...]
    )
    b_ref[...] = jnp.dot(prod_ref[...], w_bot, preferred_element_type=jnp.float32)


def _precompute(cust, prod, w1, b1):
    return pl.pallas_call(
        _precompute_body,
        out_shape=(
            jax.ShapeDtypeStruct((N_NODES, H1), jnp.float32),
            jax.ShapeDtypeStruct((N_NODES, H1), jnp.float32),
        ),
    )(cust, prod, w1, b1.reshape(1, H1))


_CPW = -(-_NCHUNK // _NW)
_PADCHUNK = _CPW * _NW


def _gather_body(a_hbm, b_hbm, row_hbm, col_hbm, out_hbm,
                 idxr, idxc, ra, rb, gt,
                 gsa0, gsb0, ws0, gsa1, gsb1, ws1):
    wid = lax.axis_index("s") * _NC + lax.axis_index("c")
    base = wid * _CPW
    cnt = jnp.minimum(_CPW, _NCHUNK - base)
    lane = lax.iota(jnp.int32, _NL)
    sems = ((gsa0, gsb0, ws0), (gsa1, gsb1, ws1))

    pltpu.sync_copy(row_hbm.at[pl.ds(base, _CPW)], idxr)
    pltpu.sync_copy(col_hbm.at[pl.ds(base, _CPW)], idxc)
    pltpu.async_copy(a_hbm.at[idxr.at[0]], ra.at[0], gsa0)
    pltpu.async_copy(b_hbm.at[idxc.at[0]], rb.at[0], gsb0)

    def pair_body(j0, carry):
        for b in (0, 1):
            j = j0 * 2 + b
            bn = 1 - b
            sa, sb, sw = sems[b]
            na, nb_, _ = sems[bn]

            @pl.when(j + 1 < cnt)
            def _():
                pltpu.async_copy(a_hbm.at[idxr.at[j + 1]], ra.at[bn], na)
                pltpu.async_copy(b_hbm.at[idxc.at[j + 1]], rb.at[bn], nb_)

            @pl.when(j < cnt)
            def _():
                pltpu.make_async_copy(a_hbm.at[idxr.at[j]], ra.at[b], sa).wait()
                pltpu.make_async_copy(b_hbm.at[idxc.at[j]], rb.at[b], sb).wait()

                @pl.when(j >= 2)
                def _():
                    pltpu.make_async_copy(
                        gt.at[b], out_hbm.at[base + j - 2], sw).wait()

                rav = ra.at[b]
                rbv = rb.at[b]
                for f in range(H1):
                    fvec = jnp.full((_NL,), f, jnp.int32)
                    for g in range(_CHUNK // _NL):
                        rows = lane + g * _NL
                        av = plsc.load_gather(rav, [rows, fvec])
                        bv = plsc.load_gather(rbv, [rows, fvec])
                        gt[b, f, pl.ds(g * _NL, _NL)] = jnp.maximum(av + bv, 0.0)
                pltpu.async_copy(gt.at[b], out_hbm.at[base + j], sw)

        return carry

    lax.fori_loop(0, _CPW // 2 + 1, pair_body, 0)
    pltpu.make_async_copy(gt.at[0], out_hbm.at[base], ws0).wait()
    pltpu.make_async_copy(gt.at[1], out_hbm.at[base], ws1).wait()


def _gather_add_relu(a_tab, b_tab, row2d, col2d):
    mesh = plsc.VectorSubcoreMesh(core_axis_name="c", subcore_axis_name="s")
    f = functools.partial(
        pl.kernel,
        mesh=mesh,
        out_type=jax.ShapeDtypeStruct((_NCHUNK, H1, _CHUNK), jnp.float32),
        compiler_params=pltpu.CompilerParams(
            use_tc_tiling_on_sc=False, needs_layout_passes=False
        ),
        scratch_types=[
            pltpu.VMEM((_CPW, _CHUNK), jnp.int32),
            pltpu.VMEM((_CPW, _CHUNK), jnp.int32),
            pltpu.VMEM((2, _CHUNK, H1), jnp.float32),
            pltpu.VMEM((2, _CHUNK, H1), jnp.float32),
            pltpu.VMEM((2, H1, _CHUNK), jnp.float32),
            pltpu.SemaphoreType.DMA,
            pltpu.SemaphoreType.DMA,
            pltpu.SemaphoreType.DMA,
            pltpu.SemaphoreType.DMA,
            pltpu.SemaphoreType.DMA,
            pltpu.SemaphoreType.DMA,
        ],
    )(_gather_body)
    return f(a_tab, b_tab, row2d, col2d)


_CB = 125


def _mlp_body(g_ref, w2_ref, b2_ref, w3_ref, b3_ref, out_ref):
    gw = jnp.concatenate([g_ref[k] for k in range(_CB)], axis=1)
    h = lax.dot_general(
        w2_ref[...], gw, (((0,), (0,)), ((), ())),
        preferred_element_type=jnp.float32,
    )
    h = jnp.maximum(h + b2_ref[...].reshape(H2, 1), 0.0)
    o = lax.dot_general(
        w3_ref[...], h, (((0,), (0,)), ((), ())),
        preferred_element_type=jnp.float32,
    ) + b3_ref[...]
    i = pl.program_id(0)
    out_ref[pl.ds(i * _CB, _CB), :] = jax.nn.sigmoid(o).reshape(_CB, _CHUNK)


def _mlp_tail(g3, w2, b2, w3, b3):
    grid = _NCHUNK // _CB
    return pl.pallas_call(
        _mlp_body,
        grid=(grid,),
        in_specs=[
            pl.BlockSpec((_CB, H1, _CHUNK), lambda i: (i, 0, 0)),
            pl.BlockSpec((H1, H2), lambda i: (0, 0)),
            pl.BlockSpec((1, H2), lambda i: (0, 0)),
            pl.BlockSpec((H2, 1), lambda i: (0, 0)),
            pl.BlockSpec((1, 1), lambda i: (0, 0)),
        ],
        out_specs=pl.BlockSpec((_NCHUNK, _CHUNK), lambda i: (0, 0)),
        out_shape=jax.ShapeDtypeStruct((_NCHUNK, _CHUNK), jnp.float32),
    )(g3, w2, b2.reshape(1, H2), w3, b3.reshape(1, 1))


def kernel(customer_emb, product_emb, edge_index, W1, b1, W2, b2, W3, b3):
    a_tab, b_tab = _precompute(customer_emb, product_emb, W1, b1)
    pad = ((0, _PADCHUNK - _NCHUNK), (0, 0))
    row2d = jnp.pad(edge_index[0].reshape(_NCHUNK, _CHUNK), pad)
    col2d = jnp.pad(edge_index[1].reshape(_NCHUNK, _CHUNK), pad)
    g3 = _gather_add_relu(a_tab, b_tab, row2d, col2d)
    out2d = _mlp_tail(g3, W2, b2, W3, b3)
    return out2d.reshape(N_EDGES)
